# baseline (speedup 1.0000x reference)
import functools

import jax
import jax.numpy as jnp
from jax.experimental import pallas as pl
from jax.experimental.pallas import tpu as pltpu

_KEEP_RATIO = 0.7
_ZSCORE_EPS = 1e-3
_EPS = 1e-6


def _fused_kernel(x_ref, rm_ref, fm_ref, rs_ref, fs_ref, out_ref, gate_ref,
                  stash_ref, m_ref, g_ref, *, bb, c, hw, b, kth, n_stash):
    p = pl.program_id(0)
    i = pl.program_id(1)
    nsteps = pl.num_programs(1)

    @pl.when(p == 0)
    def _phase0():
        xb = x_ref[...]
        m_ref[i] = jnp.sum(xb, axis=(1, 2)) * (1.0 / hw)

        @pl.when(i < n_stash)
        def _stash():
            stash_ref[pl.ds(i * bb, bb)] = xb

        @pl.when(i == nsteps - 1)
        def _gate():
            rm = rm_ref[...]
            fm = fm_ref[...]
            rs = rs_ref[...]
            fs = fs_ref[...]
            mm = m_ref[...]
            zr = jnp.abs((mm - rm) / (rs + _ZSCORE_EPS))
            zf = jnp.abs((mm - fm) / (fs + _ZSCORE_EPS))
            imp = jnp.abs(fm - rm) / (jnp.minimum(zr, zf) + _EPS)
            v = (jnp.sum(imp, axis=(0, 1), keepdims=True) * (1.0 / b))[0]

            vi = jax.lax.bitcast_convert_type(v, jnp.int32)
            hi0 = jnp.max(vi)
            lo0 = jnp.int32(-1)

            def body(_, carry):
                lo, hi = carry
                mid = lo + (hi - lo) // 2
                midf = jax.lax.bitcast_convert_type(mid, jnp.float32)
                cntm = jnp.sum((v <= midf).astype(jnp.int32))
                pred = cntm >= (kth + 1)
                return (jnp.where(pred, lo, mid), jnp.where(pred, mid, hi))

            _, hi = jax.lax.fori_loop(0, 32, body, (lo0, hi0))
            thr = jax.lax.bitcast_convert_type(hi, jnp.float32)
            grow = (v >= thr).astype(jnp.float32)
            g_ref[...] = grow
            gate_ref[...] = grow

    @pl.when(p == 1)
    def _phase1():
        g = g_ref[...]

        @pl.when(i < n_stash)
        def _from_stash():
            out_ref[...] = stash_ref[pl.ds(i * bb, bb)] * g

        @pl.when(i >= n_stash)
        def _from_hbm():
            out_ref[...] = x_ref[...] * g


def kernel(x, real_mean, fake_mean, real_std, fake_std):
    B, C, H, W = x.shape
    HW = H * W
    kth = max(0, min(int((1.0 - _KEEP_RATIO) * C), C - 1))

    xt = jnp.transpose(x, (0, 2, 3, 1))
    rm = real_mean.reshape(1, C)
    fm = fake_mean.reshape(1, C)
    rs = real_std.reshape(1, C)
    fs = fake_std.reshape(1, C)

    bb = 2
    nsteps = B // bb
    n_stash = 0

    fused = functools.partial(_fused_kernel, bb=bb, c=C, hw=HW, b=B, kth=kth,
                              n_stash=n_stash)

    def x_idx(p, i):
        return (jnp.where((p == 1) & (i < n_stash), nsteps - 1, i), 0, 0, 0)

    outt, gate = pl.pallas_call(
        fused,
        grid=(2, nsteps),
        in_specs=[
            pl.BlockSpec((bb, H, W, C), x_idx),
            pl.BlockSpec((1, C), lambda p, i: (0, 0)),
            pl.BlockSpec((1, C), lambda p, i: (0, 0)),
            pl.BlockSpec((1, C), lambda p, i: (0, 0)),
            pl.BlockSpec((1, C), lambda p, i: (0, 0)),
        ],
        out_shape=(jax.ShapeDtypeStruct((B, H, W, C), jnp.float32),
                   jax.ShapeDtypeStruct((1, C), jnp.float32)),
        out_specs=(pl.BlockSpec((bb, H, W, C),
                                lambda p, i: (jnp.where(p == 0, 0, i), 0, 0, 0)),
                   pl.BlockSpec((1, C), lambda p, i: (0, 0))),
        scratch_shapes=[
            pltpu.VMEM((n_stash * bb, H, W, C), jnp.float32),
            pltpu.VMEM((nsteps, bb, C), jnp.float32),
            pltpu.VMEM((1, C), jnp.float32),
        ],
    )(xt, rm, fm, rs, fs)

    out = jnp.transpose(outt, (0, 3, 1, 2))
    return out, gate.reshape(C)


# speedup vs baseline: 1.0099x; 1.0099x over previous
import functools

import jax
import jax.numpy as jnp
from jax.experimental import pallas as pl
from jax.experimental.pallas import tpu as pltpu

_KEEP_RATIO = 0.7
_ZSCORE_EPS = 1e-3
_EPS = 1e-6


def _fused_kernel(x_ref, rm_ref, fm_ref, rs_ref, fs_ref, out_ref, gate_ref,
                  stash_ref, m_ref, g_ref, *, bb, c, hw, b, kth, n_stash):
    p = pl.program_id(0)
    i = pl.program_id(1)
    nsteps = pl.num_programs(1)

    @pl.when(p == 0)
    def _phase0():
        xb = x_ref[...]
        m_ref[i] = jnp.sum(xb, axis=(1, 2)) * (1.0 / hw)

        @pl.when(i < n_stash)
        def _stash():
            stash_ref[pl.ds(i * bb, bb)] = xb

        @pl.when(i == nsteps - 1)
        def _gate():
            rm = rm_ref[...]
            fm = fm_ref[...]
            rs = rs_ref[...]
            fs = fs_ref[...]
            mm = m_ref[...]
            zr = jnp.abs((mm - rm) / (rs + _ZSCORE_EPS))
            zf = jnp.abs((mm - fm) / (fs + _ZSCORE_EPS))
            imp = jnp.abs(fm - rm) / (jnp.minimum(zr, zf) + _EPS)
            v = (jnp.sum(imp, axis=(0, 1), keepdims=True) * (1.0 / b))[0]

            vi = jax.lax.bitcast_convert_type(v, jnp.int32)
            hi0 = jnp.max(vi)
            lo0 = jnp.int32(-1)

            def body(_, carry):
                lo, hi = carry
                mid = lo + (hi - lo) // 2
                midf = jax.lax.bitcast_convert_type(mid, jnp.float32)
                cntm = jnp.sum((v <= midf).astype(jnp.int32))
                pred = cntm >= (kth + 1)
                return (jnp.where(pred, lo, mid), jnp.where(pred, mid, hi))

            _, hi = jax.lax.fori_loop(0, 32, body, (lo0, hi0))
            thr = jax.lax.bitcast_convert_type(hi, jnp.float32)
            grow = (v >= thr).astype(jnp.float32)
            g_ref[...] = grow
            gate_ref[...] = grow

    @pl.when(p == 1)
    def _phase1():
        g = g_ref[...]

        @pl.when(i < n_stash)
        def _from_stash():
            out_ref[...] = stash_ref[pl.ds(i * bb, bb)] * g

        @pl.when(i >= n_stash)
        def _from_hbm():
            out_ref[...] = x_ref[...] * g


def kernel(x, real_mean, fake_mean, real_std, fake_std):
    B, C, H, W = x.shape
    HW = H * W
    kth = max(0, min(int((1.0 - _KEEP_RATIO) * C), C - 1))

    xt = jnp.transpose(x, (0, 2, 3, 1))
    rm = real_mean.reshape(1, C)
    fm = fake_mean.reshape(1, C)
    rs = real_std.reshape(1, C)
    fs = fake_std.reshape(1, C)

    bb = 1
    nsteps = B // bb
    n_stash = 28

    fused = functools.partial(_fused_kernel, bb=bb, c=C, hw=HW, b=B, kth=kth,
                              n_stash=n_stash)

    def x_idx(p, i):
        return (jnp.where((p == 1) & (i < n_stash), nsteps - 1, i), 0, 0, 0)

    outt, gate = pl.pallas_call(
        fused,
        grid=(2, nsteps),
        in_specs=[
            pl.BlockSpec((bb, H, W, C), x_idx),
            pl.BlockSpec((1, C), lambda p, i: (0, 0)),
            pl.BlockSpec((1, C), lambda p, i: (0, 0)),
            pl.BlockSpec((1, C), lambda p, i: (0, 0)),
            pl.BlockSpec((1, C), lambda p, i: (0, 0)),
        ],
        out_shape=(jax.ShapeDtypeStruct((B, H, W, C), jnp.float32),
                   jax.ShapeDtypeStruct((1, C), jnp.float32)),
        out_specs=(pl.BlockSpec((bb, H, W, C),
                                lambda p, i: (jnp.where(p == 0, 0, i), 0, 0, 0)),
                   pl.BlockSpec((1, C), lambda p, i: (0, 0))),
        scratch_shapes=[
            pltpu.VMEM((n_stash * bb, H, W, C), jnp.float32),
            pltpu.VMEM((nsteps, bb, C), jnp.float32),
            pltpu.VMEM((1, C), jnp.float32),
        ],
    )(xt, rm, fm, rs, fs)

    out = jnp.transpose(outt, (0, 3, 1, 2))
    return out, gate.reshape(C)


# speedup vs baseline: 1.1428x; 1.1316x over previous
import functools

import jax
import jax.numpy as jnp
from jax.experimental import pallas as pl
from jax.experimental.pallas import tpu as pltpu

_KEEP_RATIO = 0.7
_ZSCORE_EPS = 1e-3
_EPS = 1e-6


def _fused_kernel(x_ref, rm_ref, fm_ref, rs_ref, fs_ref, out_ref, gate_ref,
                  stash_ref, m_ref, g_ref, *, bb, c, hw, b, kth, n_stash):
    p = pl.program_id(0)
    i = pl.program_id(1)
    nsteps = pl.num_programs(1)

    @pl.when(p == 0)
    def _phase0():
        xb = x_ref[...]
        m_ref[i] = jnp.sum(xb, axis=(1, 2)) * (1.0 / hw)

        @pl.when(i < n_stash)
        def _stash():
            stash_ref[pl.ds(i * bb, bb)] = xb

        @pl.when(i == nsteps - 1)
        def _gate():
            rm = rm_ref[...]
            fm = fm_ref[...]
            rs = rs_ref[...]
            fs = fs_ref[...]
            mm = m_ref[...]
            zr = jnp.abs((mm - rm) / (rs + _ZSCORE_EPS))
            zf = jnp.abs((mm - fm) / (fs + _ZSCORE_EPS))
            imp = jnp.abs(fm - rm) / (jnp.minimum(zr, zf) + _EPS)
            v = (jnp.sum(imp, axis=(0, 1), keepdims=True) * (1.0 / b))[0]

            vi = jax.lax.bitcast_convert_type(v, jnp.int32)
            hi0 = jnp.max(vi)
            lo0 = jnp.int32(-1)

            def body(_, carry):
                lo, hi = carry
                mid = lo + (hi - lo) // 2
                midf = jax.lax.bitcast_convert_type(mid, jnp.float32)
                cntm = jnp.sum((v <= midf).astype(jnp.int32))
                pred = cntm >= (kth + 1)
                return (jnp.where(pred, lo, mid), jnp.where(pred, mid, hi))

            _, hi = jax.lax.fori_loop(0, 32, body, (lo0, hi0))
            thr = jax.lax.bitcast_convert_type(hi, jnp.float32)
            grow = (v >= thr).astype(jnp.float32)
            g_ref[...] = grow
            gate_ref[...] = grow

    @pl.when(p == 1)
    def _phase1():
        g = g_ref[...]
        stride = nsteps // (nsteps - n_stash)

        @pl.when(i % stride != stride - 1)
        def _from_stash():
            out_ref[...] = stash_ref[pl.ds((i - i // stride) * bb, bb)] * g

        @pl.when(i % stride == stride - 1)
        def _from_hbm():
            out_ref[...] = x_ref[...] * g


def kernel(x, real_mean, fake_mean, real_std, fake_std):
    B, C, H, W = x.shape
    HW = H * W
    kth = max(0, min(int((1.0 - _KEEP_RATIO) * C), C - 1))

    xt = jnp.transpose(x, (0, 2, 3, 1))
    rm = real_mean.reshape(1, C)
    fm = fake_mean.reshape(1, C)
    rs = real_std.reshape(1, C)
    fs = fake_std.reshape(1, C)

    bb = 2
    nsteps = B // bb
    n_stash = 12

    fused = functools.partial(_fused_kernel, bb=bb, c=C, hw=HW, b=B, kth=kth,
                              n_stash=n_stash)

    stride = nsteps // (nsteps - n_stash)

    def x_idx(p, i):
        return (jnp.where(p == 1, n_stash + i // stride, i), 0, 0, 0)

    def out_idx(p, i):
        blk = jnp.where(i % stride == stride - 1,
                        n_stash + i // stride, i - i // stride)
        return (jnp.where(p == 0, 0, blk), 0, 0, 0)

    outt, gate = pl.pallas_call(
        fused,
        grid=(2, nsteps),
        in_specs=[
            pl.BlockSpec((bb, H, W, C), x_idx),
            pl.BlockSpec((1, C), lambda p, i: (0, 0)),
            pl.BlockSpec((1, C), lambda p, i: (0, 0)),
            pl.BlockSpec((1, C), lambda p, i: (0, 0)),
            pl.BlockSpec((1, C), lambda p, i: (0, 0)),
        ],
        out_shape=(jax.ShapeDtypeStruct((B, H, W, C), jnp.float32),
                   jax.ShapeDtypeStruct((1, C), jnp.float32)),
        out_specs=(pl.BlockSpec((bb, H, W, C), out_idx),
                   pl.BlockSpec((1, C), lambda p, i: (0, 0))),
        scratch_shapes=[
            pltpu.VMEM((n_stash * bb, H, W, C), jnp.float32),
            pltpu.VMEM((nsteps, bb, C), jnp.float32),
            pltpu.VMEM((1, C), jnp.float32),
        ],
    )(xt, rm, fm, rs, fs)

    out = jnp.transpose(outt, (0, 3, 1, 2))
    return out, gate.reshape(C)


# speedup vs baseline: 1.2396x; 1.0847x over previous
import functools

import jax
import jax.numpy as jnp
from jax.experimental import pallas as pl
from jax.experimental.pallas import tpu as pltpu

_KEEP_RATIO = 0.7
_ZSCORE_EPS = 1e-3
_EPS = 1e-6


def _fused_kernel(x_ref, rm_ref, fm_ref, rs_ref, fs_ref, out_ref, gate_ref,
                  stash_ref, m_ref, g_ref, *, bb, c, hw, b, kth, n_stash):
    p = pl.program_id(0)
    i = pl.program_id(1)
    nsteps = pl.num_programs(1)

    @pl.when(p == 0)
    def _phase0():
        xb = x_ref[...]
        m_ref[i] = jnp.sum(xb, axis=(1, 2)) * (1.0 / hw)

        @pl.when(i < n_stash)
        def _stash():
            stash_ref[pl.ds(i * bb, bb)] = xb

        @pl.when(i == nsteps - 1)
        def _gate():
            rm = rm_ref[...]
            fm = fm_ref[...]
            rs = rs_ref[...]
            fs = fs_ref[...]
            mm = m_ref[...]
            zr = jnp.abs((mm - rm) / (rs + _ZSCORE_EPS))
            zf = jnp.abs((mm - fm) / (fs + _ZSCORE_EPS))
            imp = jnp.abs(fm - rm) / (jnp.minimum(zr, zf) + _EPS)
            v = (jnp.sum(imp, axis=(0, 1), keepdims=True) * (1.0 / b))[0]

            vi = jax.lax.bitcast_convert_type(v, jnp.int32)
            hi0 = jnp.max(vi)
            lo0 = jnp.int32(-1)

            def body(_, carry):
                lo, hi = carry
                mid = lo + (hi - lo) // 2
                midf = jax.lax.bitcast_convert_type(mid, jnp.float32)
                cntm = jnp.sum((v <= midf).astype(jnp.int32))
                pred = cntm >= (kth + 1)
                return (jnp.where(pred, lo, mid), jnp.where(pred, mid, hi))

            _, hi = jax.lax.fori_loop(0, 32, body, (lo0, hi0))
            thr = jax.lax.bitcast_convert_type(hi, jnp.float32)
            grow = (v >= thr).astype(jnp.float32)
            g_ref[...] = grow
            gate_ref[...] = grow

    @pl.when(p == 1)
    def _phase1():
        g = g_ref[...]

        @pl.when(i < n_stash)
        def _from_stash():
            out_ref[...] = stash_ref[pl.ds(i * bb, bb)] * g

        @pl.when(i >= n_stash)
        def _from_hbm():
            out_ref[...] = x_ref[...] * g


def kernel(x, real_mean, fake_mean, real_std, fake_std):
    B, C, H, W = x.shape
    HW = H * W
    kth = max(0, min(int((1.0 - _KEEP_RATIO) * C), C - 1))

    xt = jnp.transpose(x, (0, 2, 3, 1))
    rm = real_mean.reshape(1, C)
    fm = fake_mean.reshape(1, C)
    rs = real_std.reshape(1, C)
    fs = fake_std.reshape(1, C)

    bb = 2
    nsteps = B // bb
    n_stash = 13

    fused = functools.partial(_fused_kernel, bb=bb, c=C, hw=HW, b=B, kth=kth,
                              n_stash=n_stash)

    def x_idx(p, i):
        return (jnp.where((p == 1) & (i < n_stash), nsteps - 1, i), 0, 0, 0)

    def out_idx(p, i):
        return (jnp.where(p == 0, 0, i), 0, 0, 0)

    outt, gate = pl.pallas_call(
        fused,
        grid=(2, nsteps),
        in_specs=[
            pl.BlockSpec((bb, H, W, C), x_idx),
            pl.BlockSpec((1, C), lambda p, i: (0, 0)),
            pl.BlockSpec((1, C), lambda p, i: (0, 0)),
            pl.BlockSpec((1, C), lambda p, i: (0, 0)),
            pl.BlockSpec((1, C), lambda p, i: (0, 0)),
        ],
        out_shape=(jax.ShapeDtypeStruct((B, H, W, C), jnp.float32),
                   jax.ShapeDtypeStruct((1, C), jnp.float32)),
        out_specs=(pl.BlockSpec((bb, H, W, C), out_idx),
                   pl.BlockSpec((1, C), lambda p, i: (0, 0))),
        scratch_shapes=[
            pltpu.VMEM((n_stash * bb, H, W, C), jnp.float32),
            pltpu.VMEM((nsteps, bb, C), jnp.float32),
            pltpu.VMEM((1, C), jnp.float32),
        ],
        compiler_params=pltpu.CompilerParams(
            vmem_limit_bytes=63 * 1024 * 1024),
    )(xt, rm, fm, rs, fs)

    out = jnp.transpose(outt, (0, 3, 1, 2))
    return out, gate.reshape(C)


# speedup vs baseline: 1.3112x; 1.0578x over previous
import functools

import jax
import jax.numpy as jnp
from jax.experimental import pallas as pl
from jax.experimental.pallas import tpu as pltpu

_KEEP_RATIO = 0.7
_ZSCORE_EPS = 1e-3
_EPS = 1e-6


def _fused_kernel(x_ref, rm_ref, fm_ref, rs_ref, fs_ref, out_ref, gate_ref,
                  stash_ref, m_ref, g_ref, *, bb, c, hw, b, kth, n_stash):
    p = pl.program_id(0)
    i = pl.program_id(1)
    nsteps = pl.num_programs(1)

    @pl.when(p == 0)
    def _phase0():
        xb = x_ref[...]
        m_ref[i] = jnp.sum(xb, axis=(1, 2)) * (1.0 / hw)

        @pl.when(i < n_stash)
        def _stash():
            stash_ref[pl.ds(i * bb, bb)] = xb

        @pl.when(i == nsteps - 1)
        def _gate():
            rm = rm_ref[...]
            fm = fm_ref[...]
            rs = rs_ref[...]
            fs = fs_ref[...]
            mm = m_ref[...]
            zr = jnp.abs((mm - rm) / (rs + _ZSCORE_EPS))
            zf = jnp.abs((mm - fm) / (fs + _ZSCORE_EPS))
            imp = jnp.abs(fm - rm) / (jnp.minimum(zr, zf) + _EPS)
            v = (jnp.sum(imp, axis=(0, 1), keepdims=True) * (1.0 / b))[0]

            vt = jnp.transpose(v)
            a = (jnp.broadcast_to(vt, (c, c)) <= jnp.broadcast_to(v, (c, c)))
            cnt = jnp.sum(a.astype(jnp.float32), axis=0, keepdims=True)
            grow = (cnt >= float(kth + 1)).astype(jnp.float32)
            g_ref[...] = grow
            gate_ref[...] = grow

    @pl.when(p == 1)
    def _phase1():
        g = g_ref[...]

        @pl.when(i < n_stash)
        def _from_stash():
            out_ref[...] = stash_ref[pl.ds(i * bb, bb)] * g

        @pl.when(i >= n_stash)
        def _from_hbm():
            out_ref[...] = x_ref[...] * g


def kernel(x, real_mean, fake_mean, real_std, fake_std):
    B, C, H, W = x.shape
    HW = H * W
    kth = max(0, min(int((1.0 - _KEEP_RATIO) * C), C - 1))

    xt = jnp.transpose(x, (0, 2, 3, 1))
    rm = real_mean.reshape(1, C)
    fm = fake_mean.reshape(1, C)
    rs = real_std.reshape(1, C)
    fs = fake_std.reshape(1, C)

    bb = 2
    nsteps = B // bb
    n_stash = 13

    fused = functools.partial(_fused_kernel, bb=bb, c=C, hw=HW, b=B, kth=kth,
                              n_stash=n_stash)

    def x_idx(p, i):
        return (jnp.where((p == 1) & (i < n_stash), nsteps - 1, i), 0, 0, 0)

    def out_idx(p, i):
        return (jnp.where(p == 0, 0, i), 0, 0, 0)

    outt, gate = pl.pallas_call(
        fused,
        grid=(2, nsteps),
        in_specs=[
            pl.BlockSpec((bb, H, W, C), x_idx),
            pl.BlockSpec((1, C), lambda p, i: (0, 0)),
            pl.BlockSpec((1, C), lambda p, i: (0, 0)),
            pl.BlockSpec((1, C), lambda p, i: (0, 0)),
            pl.BlockSpec((1, C), lambda p, i: (0, 0)),
        ],
        out_shape=(jax.ShapeDtypeStruct((B, H, W, C), jnp.float32),
                   jax.ShapeDtypeStruct((1, C), jnp.float32)),
        out_specs=(pl.BlockSpec((bb, H, W, C), out_idx),
                   pl.BlockSpec((1, C), lambda p, i: (0, 0))),
        scratch_shapes=[
            pltpu.VMEM((n_stash * bb, H, W, C), jnp.float32),
            pltpu.VMEM((nsteps, bb, C), jnp.float32),
            pltpu.VMEM((1, C), jnp.float32),
        ],
        compiler_params=pltpu.CompilerParams(
            vmem_limit_bytes=63 * 1024 * 1024),
    )(xt, rm, fm, rs, fs)

    out = jnp.transpose(outt, (0, 3, 1, 2))
    return out, gate.reshape(C)
